# Initial kernel scaffold; baseline (speedup 1.0000x reference)
import jax
import jax.numpy as jnp
from jax.experimental import pallas as pl


def kernel(token_ids, table):
    raise NotImplementedError("write your pallas kernel here")


# speedup vs baseline: 4.9458x; 4.9458x over previous
import functools

import jax
import jax.numpy as jnp
from jax import lax
from jax.experimental import pallas as pl
from jax.experimental.pallas import tpu as pltpu
from jax.experimental.pallas import tpu_sc as plsc

EMBED_DIM = 32
FLAT_B = 16384 * 200
NUM_WORKERS = 32
PER_WORKER = FLAT_B // NUM_WORKERS
CHUNK = 2048
NUM_CHUNKS = PER_WORKER // CHUNK

_mesh = plsc.VectorSubcoreMesh(core_axis_name="c", subcore_axis_name="s")


@functools.partial(
    pl.kernel,
    mesh=_mesh,
    out_type=jax.ShapeDtypeStruct((FLAT_B, EMBED_DIM), jnp.float32),
    scratch_types=[
        pltpu.VMEM((CHUNK,), jnp.int32),
        pltpu.VMEM((CHUNK, EMBED_DIM), jnp.float32),
        pltpu.SemaphoreType.DMA,
    ],
    compiler_params=pltpu.CompilerParams(use_tc_tiling_on_sc=False),
)
def _gather_rows(idx_hbm, table_hbm, out_hbm, idx_v, rows_v, sem):
    wid = lax.axis_index("s") * 2 + lax.axis_index("c")
    base = wid * PER_WORKER

    def body(i, carry):
        off = base + i * CHUNK
        pltpu.sync_copy(idx_hbm.at[pl.ds(off, CHUNK)], idx_v)
        pltpu.async_copy(table_hbm.at[idx_v], rows_v, sem).wait()
        pltpu.sync_copy(rows_v, out_hbm.at[pl.ds(off, CHUNK)])
        return carry

    lax.fori_loop(0, NUM_CHUNKS, body, 0)


def kernel(token_ids, table):
    idx = token_ids.reshape(-1).astype(jnp.int32)
    out = _gather_rows(idx, table)
    return out.reshape(token_ids.shape + (EMBED_DIM,))


# speedup vs baseline: 4.9814x; 1.0072x over previous
import functools

import jax
import jax.numpy as jnp
from jax import lax
from jax.experimental import pallas as pl
from jax.experimental.pallas import tpu as pltpu
from jax.experimental.pallas import tpu_sc as plsc

EMBED_DIM = 32
FLAT_B = 16384 * 200
NUM_WORKERS = 32
PER_WORKER = FLAT_B // NUM_WORKERS
CHUNK = 1600
NUM_CHUNKS = PER_WORKER // CHUNK
NBUF = 2
N_OUTER = NUM_CHUNKS // NBUF

_mesh = plsc.VectorSubcoreMesh(core_axis_name="c", subcore_axis_name="s")


@functools.partial(
    pl.kernel,
    mesh=_mesh,
    out_type=jax.ShapeDtypeStruct((FLAT_B, EMBED_DIM), jnp.float32),
    scratch_types=(
        [pltpu.VMEM((CHUNK,), jnp.int32)] * NBUF
        + [pltpu.VMEM((CHUNK, EMBED_DIM), jnp.float32)] * NBUF
        + [pltpu.SemaphoreType.DMA] * (2 * NBUF)
    ),
    compiler_params=pltpu.CompilerParams(use_tc_tiling_on_sc=False),
)
def _gather_rows(idx_hbm, table_hbm, out_hbm, *scr):
    idxs = scr[0:NBUF]
    rows = scr[NBUF:2 * NBUF]
    gsems = scr[2 * NBUF:3 * NBUF]
    wsems = scr[3 * NBUF:4 * NBUF]

    wid = lax.axis_index("s") * 2 + lax.axis_index("c")
    base = wid * PER_WORKER

    def start_gather(chunk_id, b):
        off = base + chunk_id * CHUNK
        pltpu.sync_copy(idx_hbm.at[pl.ds(off, CHUNK)], idxs[b])
        pltpu.async_copy(table_hbm.at[idxs[b]], rows[b], gsems[b])

    def wait_gather(b):
        pltpu.make_async_copy(table_hbm.at[idxs[b]], rows[b], gsems[b]).wait()

    def start_write(chunk_off, b):
        pltpu.async_copy(rows[b], out_hbm.at[pl.ds(chunk_off, CHUNK)], wsems[b])

    def wait_write(b):
        pltpu.make_async_copy(rows[b], out_hbm.at[pl.ds(base, CHUNK)], wsems[b]).wait()

    for b in range(NBUF):
        start_gather(b, b)

    def outer(g, carry):
        for b in range(NBUF):
            c = g * NBUF + b
            wait_gather(b)
            start_write(base + c * CHUNK, b)
            wait_write(b)
            start_gather(c + NBUF, b)
        return carry

    lax.fori_loop(0, N_OUTER - 1, outer, 0)

    for b in range(NBUF):
        c = (N_OUTER - 1) * NBUF + b
        wait_gather(b)
        start_write(base + c * CHUNK, b)
    for b in range(NBUF):
        wait_write(b)


def kernel(token_ids, table):
    idx = token_ids.reshape(-1).astype(jnp.int32)
    out = _gather_rows(idx, table)
    return out.reshape(token_ids.shape + (EMBED_DIM,))
